# Initial kernel scaffold; baseline (speedup 1.0000x reference)
import jax
import jax.numpy as jnp
from jax.experimental import pallas as pl


def kernel(hidden, route_weights, W1, b1, W2, b2):
    raise NotImplementedError("write your pallas kernel here")


# speedup vs baseline: 1.1634x; 1.1634x over previous
import functools

import jax
import jax.numpy as jnp
from jax.experimental import pallas as pl


def _moe_body(hf_ref, route_ref, w1_ref, b1_ref, w2_ref, b2_ref, out_ref,
              *, n_experts, eps):
    e = pl.program_id(0)
    f = pl.program_id(1)

    r = route_ref[...]
    t = r.shape[0]
    iota_e = jax.lax.broadcasted_iota(jnp.int32, r.shape, 1)

    m0 = jnp.max(r, axis=1, keepdims=True)
    idx0 = jnp.min(jnp.where(r == m0, iota_e, n_experts), axis=1,
                   keepdims=True)
    r2 = jnp.where(iota_e == idx0, -jnp.inf, r)
    m1 = jnp.max(r2, axis=1, keepdims=True)
    idx1 = jnp.min(jnp.where(r2 == m1, iota_e, n_experts), axis=1,
                   keepdims=True)
    denom = jnp.maximum(m0 + m1, eps)
    w0 = m0 / denom
    w1 = m1 / denom
    gate = (jnp.where(idx0 == e, w0, 0.0) + jnp.where(idx1 == e, w1, 0.0))

    x = hf_ref[...].astype(jnp.bfloat16)
    w1b = w1_ref[0].astype(jnp.bfloat16)
    h = jnp.dot(x, w1b, preferred_element_type=jnp.float32)
    h = h + b1_ref[0]
    h = jax.nn.gelu(h)
    y = jnp.dot(h.astype(jnp.bfloat16), w2_ref[0].astype(jnp.bfloat16),
                preferred_element_type=jnp.float32)
    contrib = gate * y

    @pl.when(f == 0)
    def _():
        contrib_b2 = contrib + gate * b2_ref[0]
        @pl.when(e == 0)
        def _():
            out_ref[...] = contrib_b2
        @pl.when(e != 0)
        def _():
            out_ref[...] += contrib_b2

    @pl.when(f != 0)
    def _():
        out_ref[...] += contrib


def kernel(hidden, route_weights, W1, b1, W2, b2):
    b, s, d = hidden.shape
    t = b * s
    e, _, ff = W1.shape
    hf = hidden.reshape(t, d)
    rf = route_weights.reshape(t, e)

    fft = 512
    if ff % fft != 0:
        fft = ff
    nff = ff // fft

    grid = (e, nff)
    out = pl.pallas_call(
        functools.partial(_moe_body, n_experts=e, eps=1e-9),
        grid=grid,
        in_specs=[
            pl.BlockSpec((t, d), lambda ei, fi: (0, 0)),
            pl.BlockSpec((t, e), lambda ei, fi: (0, 0)),
            pl.BlockSpec((1, d, fft), lambda ei, fi: (ei, 0, fi)),
            pl.BlockSpec((1, 1, fft), lambda ei, fi: (ei, 0, fi)),
            pl.BlockSpec((1, fft, d), lambda ei, fi: (ei, fi, 0)),
            pl.BlockSpec((1, 1, d), lambda ei, fi: (ei, 0, 0)),
        ],
        out_specs=pl.BlockSpec((t, d), lambda ei, fi: (0, 0)),
        out_shape=jax.ShapeDtypeStruct((t, d), jnp.float32),
    )(hf, rf, W1, b1.reshape(e, 1, ff), W2, b2.reshape(e, 1, d))
    return out.reshape(b, s, d)


# speedup vs baseline: 1.4394x; 1.2372x over previous
import functools

import jax
import jax.numpy as jnp
from jax.experimental import pallas as pl


def _moe_body(hf_ref, route_ref, w1_ref, b1_ref, w2_ref, b2_ref, out_ref,
              *, n_experts, eps):
    e = pl.program_id(0)
    f = pl.program_id(1)

    r = route_ref[...]
    t = r.shape[0]
    iota_e = jax.lax.broadcasted_iota(jnp.int32, r.shape, 1)

    m0 = jnp.max(r, axis=1, keepdims=True)
    idx0 = jnp.min(jnp.where(r == m0, iota_e, n_experts), axis=1,
                   keepdims=True)
    r2 = jnp.where(iota_e == idx0, -jnp.inf, r)
    m1 = jnp.max(r2, axis=1, keepdims=True)
    idx1 = jnp.min(jnp.where(r2 == m1, iota_e, n_experts), axis=1,
                   keepdims=True)
    denom = jnp.maximum(m0 + m1, eps)
    w0 = m0 / denom
    w1 = m1 / denom
    gate = (jnp.where(idx0 == e, w0, 0.0) + jnp.where(idx1 == e, w1, 0.0))

    x = hf_ref[...].astype(jnp.bfloat16)
    w1b = w1_ref[0].astype(jnp.bfloat16)
    h = jnp.dot(x, w1b, preferred_element_type=jnp.float32)
    h = h + b1_ref[0]
    h = jax.nn.gelu(h)
    y = jnp.dot(h.astype(jnp.bfloat16), w2_ref[0].astype(jnp.bfloat16),
                preferred_element_type=jnp.float32)
    contrib = gate * y

    @pl.when(f == 0)
    def _():
        contrib_b2 = contrib + gate * b2_ref[0]
        @pl.when(e == 0)
        def _():
            out_ref[...] = contrib_b2
        @pl.when(e != 0)
        def _():
            out_ref[...] += contrib_b2

    @pl.when(f != 0)
    def _():
        out_ref[...] += contrib


def kernel(hidden, route_weights, W1, b1, W2, b2):
    b, s, d = hidden.shape
    t = b * s
    e, _, ff = W1.shape
    hf = hidden.reshape(t, d)
    rf = route_weights.reshape(t, e)

    fft = 1024
    if ff % fft != 0:
        fft = ff
    nff = ff // fft

    grid = (e, nff)
    out = pl.pallas_call(
        functools.partial(_moe_body, n_experts=e, eps=1e-9),
        grid=grid,
        in_specs=[
            pl.BlockSpec((t, d), lambda ei, fi: (0, 0)),
            pl.BlockSpec((t, e), lambda ei, fi: (0, 0)),
            pl.BlockSpec((1, d, fft), lambda ei, fi: (ei, 0, fi)),
            pl.BlockSpec((1, 1, fft), lambda ei, fi: (ei, 0, fi)),
            pl.BlockSpec((1, fft, d), lambda ei, fi: (ei, fi, 0)),
            pl.BlockSpec((1, 1, d), lambda ei, fi: (ei, 0, 0)),
        ],
        out_specs=pl.BlockSpec((t, d), lambda ei, fi: (0, 0)),
        out_shape=jax.ShapeDtypeStruct((t, d), jnp.float32),
    )(hf, rf, W1, b1.reshape(e, 1, ff), W2, b2.reshape(e, 1, d))
    return out.reshape(b, s, d)


# speedup vs baseline: 2.1276x; 1.4781x over previous
import functools

import jax
import jax.numpy as jnp
from jax import lax
from jax.experimental import pallas as pl
from jax.experimental.pallas import tpu as pltpu
from jax.experimental.pallas import tpu_sc as plsc

_NC, _NS = 2, 16
_NW = _NC * _NS
_BT = 128


def _icumsum(x, axis):
    n = x.shape[axis]
    shift = 1
    while shift < n:
        rolled = jnp.roll(x, shift, axis=axis)
        mask = lax.broadcasted_iota(jnp.int32, x.shape, axis) >= shift
        x = x + jnp.where(mask, rolled, jnp.zeros_like(x))
        shift *= 2
    return x


def _route_body(r_ref, pos0_ref, pos1_ref, w0_ref, w1_ref, blk_ref, nbu_ref,
                *, n_e, n_t, nb, eps):
    r = r_ref[...]
    iota_e = lax.broadcasted_iota(jnp.int32, (n_e, n_t), 0)
    m0 = jnp.max(r, axis=0, keepdims=True)
    e0 = jnp.min(jnp.where(r == m0, iota_e, n_e), axis=0, keepdims=True)
    r2 = jnp.where(iota_e == e0, -jnp.inf, r)
    m1 = jnp.max(r2, axis=0, keepdims=True)
    e1 = jnp.min(jnp.where(r2 == m1, iota_e, n_e), axis=0, keepdims=True)
    denom = jnp.maximum(m0 + m1, eps)
    w0_ref[...] = m0 / denom
    w1_ref[...] = m1 / denom

    oh0 = (iota_e == e0).astype(jnp.int32)
    oh1 = (iota_e == e1).astype(jnp.int32)
    ohs = oh0 + oh1
    csum = _icumsum(ohs, axis=1)
    excl = csum - ohs
    rank0 = jnp.sum(oh0 * excl, axis=0, keepdims=True)
    rank1 = jnp.sum(oh1 * excl, axis=0, keepdims=True)

    counts = csum[:, n_t - 1:n_t]
    padded = ((counts + _BT - 1) // _BT) * _BT
    off_incl = _icumsum(padded, axis=0)
    off = off_incl - padded
    pos0_ref[...] = jnp.sum(oh0 * off, axis=0, keepdims=True) + rank0
    pos1_ref[...] = jnp.sum(oh1 * off, axis=0, keepdims=True) + rank1

    istart = lax.broadcasted_iota(jnp.int32, (1, nb), 1) * _BT
    blk_ref[...] = jnp.sum((istart >= off).astype(jnp.int32), axis=0,
                           keepdims=True) - 1
    nbu_ref[...] = off_incl[n_e - 1:n_e, :] // _BT


def _routing(r8, n_e, n_t, nb, eps=1e-9):
    outs = pl.pallas_call(
        functools.partial(_route_body, n_e=n_e, n_t=n_t, nb=nb, eps=eps),
        grid=(1,),
        in_specs=[pl.BlockSpec((n_e, n_t), lambda i: (0, 0))],
        out_specs=[
            pl.BlockSpec((1, n_t), lambda i: (0, 0)),
            pl.BlockSpec((1, n_t), lambda i: (0, 0)),
            pl.BlockSpec((1, n_t), lambda i: (0, 0)),
            pl.BlockSpec((1, n_t), lambda i: (0, 0)),
            pl.BlockSpec((1, nb), lambda i: (0, 0)),
            pl.BlockSpec((1, 1), lambda i: (0, 0)),
        ],
        out_shape=[
            jax.ShapeDtypeStruct((1, n_t), jnp.int32),
            jax.ShapeDtypeStruct((1, n_t), jnp.int32),
            jax.ShapeDtypeStruct((1, n_t), jnp.float32),
            jax.ShapeDtypeStruct((1, n_t), jnp.float32),
            jax.ShapeDtypeStruct((1, nb), jnp.int32),
            jax.ShapeDtypeStruct((1, 1), jnp.int32),
        ],
    )(r8)
    return outs


def _sc_scatter(hf, p0, p1, n_pack):
    n_t, d = hf.shape
    chunk = n_t // _NW
    mesh = plsc.VectorSubcoreMesh(core_axis_name="c", subcore_axis_name="s")

    @functools.partial(
        pl.kernel, mesh=mesh,
        out_type=jax.ShapeDtypeStruct((n_pack, d), jnp.float32),
        scratch_types=[
            pltpu.VMEM((chunk,), jnp.int32),
            pltpu.VMEM((chunk,), jnp.int32),
            pltpu.VMEM((chunk, d), jnp.float32),
        ],
    )
    def k(hf_hbm, p0_hbm, p1_hbm, packed_hbm, i0_v, i1_v, rows_v):
        wid = lax.axis_index("s") * _NC + lax.axis_index("c")
        base = wid * chunk
        pltpu.sync_copy(p0_hbm.at[pl.ds(base, chunk)], i0_v)
        pltpu.sync_copy(p1_hbm.at[pl.ds(base, chunk)], i1_v)
        pltpu.sync_copy(hf_hbm.at[pl.ds(base, chunk)], rows_v)
        pltpu.sync_copy(rows_v, packed_hbm.at[i0_v])
        pltpu.sync_copy(rows_v, packed_hbm.at[i1_v])

    return k(hf, p0, p1)


def _sc_gather(packed_out, p0, p1):
    n_t = p0.shape[0]
    d = packed_out.shape[1]
    chunk = n_t // _NW
    mesh = plsc.VectorSubcoreMesh(core_axis_name="c", subcore_axis_name="s")

    @functools.partial(
        pl.kernel, mesh=mesh,
        out_type=[jax.ShapeDtypeStruct((n_t, d), jnp.float32),
                  jax.ShapeDtypeStruct((n_t, d), jnp.float32)],
        scratch_types=[
            pltpu.VMEM((chunk,), jnp.int32),
            pltpu.VMEM((chunk,), jnp.int32),
            pltpu.VMEM((chunk, d), jnp.float32),
            pltpu.VMEM((chunk, d), jnp.float32),
        ],
    )
    def k(po_hbm, p0_hbm, p1_hbm, g0_hbm, g1_hbm, i0_v, i1_v, r0_v, r1_v):
        wid = lax.axis_index("s") * _NC + lax.axis_index("c")
        base = wid * chunk
        pltpu.sync_copy(p0_hbm.at[pl.ds(base, chunk)], i0_v)
        pltpu.sync_copy(p1_hbm.at[pl.ds(base, chunk)], i1_v)
        pltpu.sync_copy(po_hbm.at[i0_v], r0_v)
        pltpu.sync_copy(po_hbm.at[i1_v], r1_v)
        pltpu.sync_copy(r0_v, g0_hbm.at[pl.ds(base, chunk)])
        pltpu.sync_copy(r1_v, g1_hbm.at[pl.ds(base, chunk)])

    return k(packed_out, p0, p1)


def _group_body(blk_ref, nbu_ref, x_ref, w1_ref, b1_ref, w2_ref, b2_ref,
                o_ref):
    i = pl.program_id(0)

    @pl.when(i < nbu_ref[0])
    def _():
        x = x_ref[...].astype(jnp.bfloat16)
        h = jnp.dot(x, w1_ref[0].astype(jnp.bfloat16),
                    preferred_element_type=jnp.float32)
        h = jax.nn.gelu(h + b1_ref[0])
        y = jnp.dot(h.astype(jnp.bfloat16), w2_ref[0].astype(jnp.bfloat16),
                    preferred_element_type=jnp.float32)
        o_ref[...] = y + b2_ref[0]


def _grouped_mlp(packed, W1, b1, W2, b2, blk, nbu):
    n_pack, d = packed.shape
    n_e, _, ff = W1.shape
    nb = n_pack // _BT
    grid_spec = pltpu.PrefetchScalarGridSpec(
        num_scalar_prefetch=2,
        grid=(nb,),
        in_specs=[
            pl.BlockSpec((_BT, d), lambda i, blk_m, nbu_m: (i, 0)),
            pl.BlockSpec((1, d, ff), lambda i, blk_m, nbu_m: (blk_m[i], 0, 0)),
            pl.BlockSpec((1, 1, ff), lambda i, blk_m, nbu_m: (blk_m[i], 0, 0)),
            pl.BlockSpec((1, ff, d), lambda i, blk_m, nbu_m: (blk_m[i], 0, 0)),
            pl.BlockSpec((1, 1, d), lambda i, blk_m, nbu_m: (blk_m[i], 0, 0)),
        ],
        out_specs=pl.BlockSpec((_BT, d), lambda i, blk_m, nbu_m: (i, 0)),
    )
    return pl.pallas_call(
        _group_body,
        grid_spec=grid_spec,
        out_shape=jax.ShapeDtypeStruct((n_pack, d), jnp.float32),
    )(blk, nbu, packed, W1, b1.reshape(n_e, 1, ff), W2, b2.reshape(n_e, 1, d))


def _combine_body(w0_ref, w1_ref, g0_ref, g1_ref, o_ref):
    o_ref[...] = w0_ref[...] * g0_ref[...] + w1_ref[...] * g1_ref[...]


def _combine(w0c, w1c, g0, g1):
    n_t, d = g0.shape
    return pl.pallas_call(
        _combine_body,
        grid=(1,),
        in_specs=[
            pl.BlockSpec((n_t, 1), lambda i: (0, 0)),
            pl.BlockSpec((n_t, 1), lambda i: (0, 0)),
            pl.BlockSpec((n_t, d), lambda i: (0, 0)),
            pl.BlockSpec((n_t, d), lambda i: (0, 0)),
        ],
        out_specs=pl.BlockSpec((n_t, d), lambda i: (0, 0)),
        out_shape=jax.ShapeDtypeStruct((n_t, d), jnp.float32),
    )(w0c, w1c, g0, g1)


def kernel(hidden, route_weights, W1, b1, W2, b2):
    b, s, d = hidden.shape
    n_t = b * s
    n_e, _, ff = W1.shape
    assert n_t % (8 * _NW) == 0
    n_pack = n_t * 2 + n_e * _BT
    nb = n_pack // _BT

    hf = hidden.reshape(n_t, d)
    r8 = route_weights.reshape(n_t, n_e).T

    pos0, pos1, w0, w1, blk, nbu = _routing(r8, n_e, n_t, nb)
    p0 = pos0.reshape(n_t)
    p1 = pos1.reshape(n_t)

    packed = _sc_scatter(hf, p0, p1, n_pack)
    packed_out = _grouped_mlp(packed, W1, b1, W2, b2,
                              blk.reshape(nb), nbu.reshape(1))
    g0, g1 = _sc_gather(packed_out, p0, p1)
    out = _combine(w0.reshape(n_t, 1), w1.reshape(n_t, 1), g0, g1)
    return out.reshape(b, s, d)
